# baseline (speedup 1.0000x reference)
import jax
import jax.numpy as jnp
from jax.experimental import pallas as pl
from jax.experimental.pallas import tpu as pltpu

C = 256
CIM = 32
TOP = 9
HW = 9216
QBLK = 128
NBLK = HW // QBLK

_NEG = -100000.0


def _fused_body(x_ref, wq_ref, wk_ref, wv_ref, cwb_ref, pm1t_ref, pm2_ref,
                bw_ref, w1_ref, w2_ref, mlpb_ref, mlpg_ref, mlpbeta_ref,
                catt_ref, catt1_ref, dsn_ref, dsnsc_ref,
                xo_ref, r0_ref,
                qn_s, kn_s, v_s, proj_s):
    i = pl.program_id(0)

    @pl.when(i == 0)
    def _proj():
        x = x_ref[...]
        def norm_rows(w_ref):
            p = jax.lax.dot_general(w_ref[...], x, (((1,), (0,)), ((), ())),
                                    preferred_element_type=jnp.float32)
            nrm = jnp.sqrt(jnp.sum(p * p, axis=0, keepdims=True))
            return p / jnp.maximum(nrm, 1e-12)
        qn_s[...] = norm_rows(wq_ref)
        kn_s[...] = norm_rows(wk_ref)
        v_s[...] = jax.lax.dot_general(wv_ref[...], x, (((1,), (0,)), ((), ())),
                                       preferred_element_type=jnp.float32)

    sl = pl.ds(i * QBLK, QBLK)
    qn = qn_s[:, sl]
    kn = kn_s[...]
    att = jax.lax.dot_general(qn, kn, (((0,), (0,)), ((), ())),
                              preferred_element_type=jnp.float32)

    accs = [jnp.full((QBLK, 128), -2.0, jnp.float32) for _ in range(TOP)]
    for c in range(HW // 128):
        t = att[:, c * 128:(c + 1) * 128]
        for j in range(TOP):
            hi = jnp.maximum(accs[j], t)
            if j + 1 < TOP:
                t = jnp.minimum(accs[j], t)
            accs[j] = hi
    cand = jnp.concatenate(accs, axis=1)
    tops = [jnp.max(cand, axis=1, keepdims=True)]
    for j in range(1, TOP):
        t = jnp.where(cand >= tops[-1], -2.0, cand)
        tops.append(jnp.max(t, axis=1, keepdims=True))

    cw = cwb_ref[0, 0]
    cb = cwb_ref[0, 1]
    ys = [t * cw + cb for t in tops]
    h1 = jnp.zeros((QBLK, 2 * TOP), jnp.float32)
    for j in range(TOP):
        h1 = h1 + jnp.maximum(ys[j], 0.0) * pm1t_ref[j:j + 1, :]
    h1 = jnp.maximum(h1, 0.0)
    masks = []
    for j in range(TOP):
        lg = jnp.sum(h1 * pm2_ref[j:j + 1, :], axis=1, keepdims=True)
        masks.append(1.0 / (1.0 + jnp.exp(-lg)))
    ss = [jnp.where(y > 0.0, y, _NEG) for y in ys]
    smax = ss[0]
    for j in range(1, TOP):
        smax = jnp.maximum(smax, ss[j])
    es = [jnp.exp(s - smax) for s in ss]
    tot = es[0]
    for j in range(1, TOP):
        tot = tot + es[j]
    aws = [es[j] / tot * masks[j] for j in range(TOP)]
    w = jnp.zeros((QBLK, HW), jnp.float32)
    for j in range(TOP):
        cj = aws[j] - (aws[j + 1] if j + 1 < TOP else 0.0)
        w = w + jnp.where(att >= tops[j], cj, 0.0)

    out = jax.lax.dot_general(v_s[...], w, (((1,), (1,)), ((), ())),
                              preferred_element_type=jnp.float32)
    proj_s[:, sl] = jnp.dot(bw_ref[...], out,
                            preferred_element_type=jnp.float32)

    @pl.when(i == NBLK - 1)
    def _tail():
        vimg = catt_ref[...] * proj_s[...]
        f2 = catt1_ref[...] * x_ref[...]
        z = (jnp.dot(w1_ref[...], vimg, preferred_element_type=jnp.float32)
             + jnp.dot(w2_ref[...], f2, preferred_element_type=jnp.float32)
             + mlpb_ref[...])
        m = jnp.mean(z, axis=1, keepdims=True)
        var = jnp.mean((z - m) * (z - m), axis=1, keepdims=True)
        xo = jnp.maximum(
            mlpg_ref[...] * (z - m) / jnp.sqrt(var + 1e-5) + mlpbeta_ref[...],
            0.0)
        xo_ref[...] = xo
        r = (jnp.dot(dsn_ref[...], xo, preferred_element_type=jnp.float32)
             + dsnsc_ref[0, 0])
        rm = jnp.mean(r)
        rv = jnp.mean((r - rm) * (r - rm))
        r0_ref[...] = jnp.maximum(
            dsnsc_ref[0, 2] * (r - rm) / jnp.sqrt(rv + 1e-5) + dsnsc_ref[0, 1],
            0.0)


def kernel(feat, Wq, Wk, Wv, conv_w, conv_b, pm_w1, pm_w2, catt, catt1,
           back_w, mlp_w, mlp_b, mlp_g, mlp_beta, dsn_w, dsn_b, dsn_g,
           dsn_beta, H, W):
    x = feat.reshape(C, HW)
    cwb = jnp.stack([conv_w[0, 0, 0, 0], conv_b[0]]).reshape(1, 2)
    pm1t = pm_w1.T
    dsnsc = jnp.stack([dsn_b[0], dsn_beta[0], dsn_g[0]]).reshape(1, 3)

    full = lambda s: pl.BlockSpec(s, lambda i: tuple(0 for _ in s))
    xo, r0 = pl.pallas_call(
        _fused_body,
        grid=(NBLK,),
        in_specs=[
            full((C, HW)),
            full((CIM, C)),
            full((CIM, C)),
            full((CIM, C)),
            full((1, 2)),
            full((TOP, 2 * TOP)),
            full((TOP, 2 * TOP)),
            full((C, CIM)),
            full((C, C)),
            full((C, C)),
            full((C, 1)),
            full((C, 1)),
            full((C, 1)),
            full((C, 1)),
            full((C, 1)),
            full((1, C)),
            full((1, 3)),
        ],
        out_specs=[full((C, HW)), full((1, HW))],
        out_shape=[jax.ShapeDtypeStruct((C, HW), jnp.float32),
                   jax.ShapeDtypeStruct((1, HW), jnp.float32)],
        scratch_shapes=[
            pltpu.VMEM((CIM, HW), jnp.float32),
            pltpu.VMEM((CIM, HW), jnp.float32),
            pltpu.VMEM((CIM, HW), jnp.float32),
            pltpu.VMEM((C, HW), jnp.float32),
        ],
    )(x, Wq, Wk, Wv, cwb, pm1t, pm_w2, back_w, mlp_w[:, :C], mlp_w[:, C:],
      mlp_b.reshape(C, 1), mlp_g.reshape(C, 1), mlp_beta.reshape(C, 1),
      catt.reshape(C, 1), catt1.reshape(C, 1), dsn_w, dsnsc)

    h = feat.shape[2]
    w_ = feat.shape[3]
    return (xo.reshape(1, C, h, w_), r0.reshape(1, 1, h, w_))


# speedup vs baseline: 1.0380x; 1.0380x over previous
import jax
import jax.numpy as jnp
from jax.experimental import pallas as pl
from jax.experimental.pallas import tpu as pltpu

C = 256
CIM = 32
TOP = 9
HW = 9216
QBLK = 256
NBLK = HW // QBLK

_NEG = -100000.0


def _fused_body(x_ref, wq_ref, wk_ref, wv_ref, cwb_ref, pm1t_ref, pm2_ref,
                bw_ref, w1_ref, w2_ref, mlpb_ref, mlpg_ref, mlpbeta_ref,
                catt_ref, catt1_ref, dsn_ref, dsnsc_ref,
                xo_ref, r0_ref,
                qn_s, kn_s, v_s, proj_s):
    i = pl.program_id(0)

    @pl.when(i == 0)
    def _proj():
        x = x_ref[...]
        def norm_rows(w_ref):
            p = jax.lax.dot_general(w_ref[...], x, (((1,), (0,)), ((), ())),
                                    preferred_element_type=jnp.float32)
            nrm = jnp.sqrt(jnp.sum(p * p, axis=0, keepdims=True))
            return p / jnp.maximum(nrm, 1e-12)
        qn_s[...] = norm_rows(wq_ref)
        kn_s[...] = norm_rows(wk_ref)
        v_s[...] = jax.lax.dot_general(wv_ref[...], x, (((1,), (0,)), ((), ())),
                                       preferred_element_type=jnp.float32)

    sl = pl.ds(i * QBLK, QBLK)
    qn = qn_s[:, sl]
    kn = kn_s[...]
    att = jax.lax.dot_general(qn, kn, (((0,), (0,)), ((), ())),
                              preferred_element_type=jnp.float32)

    accs = [jnp.full((QBLK, 128), -2.0, jnp.float32) for _ in range(TOP)]
    for c in range(HW // 128):
        t = att[:, c * 128:(c + 1) * 128]
        for j in range(TOP):
            hi = jnp.maximum(accs[j], t)
            if j + 1 < TOP:
                t = jnp.minimum(accs[j], t)
            accs[j] = hi
    cand = jnp.concatenate(accs, axis=1)
    tops = [jnp.max(cand, axis=1, keepdims=True)]
    for j in range(1, TOP):
        t = jnp.where(cand >= tops[-1], -2.0, cand)
        tops.append(jnp.max(t, axis=1, keepdims=True))

    cw = cwb_ref[0, 0]
    cb = cwb_ref[0, 1]
    ys = [t * cw + cb for t in tops]
    h1 = jnp.zeros((QBLK, 2 * TOP), jnp.float32)
    for j in range(TOP):
        h1 = h1 + jnp.maximum(ys[j], 0.0) * pm1t_ref[j:j + 1, :]
    h1 = jnp.maximum(h1, 0.0)
    masks = []
    for j in range(TOP):
        lg = jnp.sum(h1 * pm2_ref[j:j + 1, :], axis=1, keepdims=True)
        masks.append(1.0 / (1.0 + jnp.exp(-lg)))
    ss = [jnp.where(y > 0.0, y, _NEG) for y in ys]
    smax = ss[0]
    for j in range(1, TOP):
        smax = jnp.maximum(smax, ss[j])
    es = [jnp.exp(s - smax) for s in ss]
    tot = es[0]
    for j in range(1, TOP):
        tot = tot + es[j]
    aws = [es[j] / tot * masks[j] for j in range(TOP)]
    w = jnp.zeros((QBLK, HW), jnp.float32)
    for j in range(TOP):
        cj = aws[j] - (aws[j + 1] if j + 1 < TOP else 0.0)
        w = w + jnp.where(att >= tops[j], cj, 0.0)

    out = jax.lax.dot_general(v_s[...], w, (((1,), (1,)), ((), ())),
                              preferred_element_type=jnp.float32)
    proj_s[:, sl] = jnp.dot(bw_ref[...], out,
                            preferred_element_type=jnp.float32)

    @pl.when(i == NBLK - 1)
    def _tail():
        vimg = catt_ref[...] * proj_s[...]
        f2 = catt1_ref[...] * x_ref[...]
        z = (jnp.dot(w1_ref[...], vimg, preferred_element_type=jnp.float32)
             + jnp.dot(w2_ref[...], f2, preferred_element_type=jnp.float32)
             + mlpb_ref[...])
        m = jnp.mean(z, axis=1, keepdims=True)
        var = jnp.mean((z - m) * (z - m), axis=1, keepdims=True)
        xo = jnp.maximum(
            mlpg_ref[...] * (z - m) / jnp.sqrt(var + 1e-5) + mlpbeta_ref[...],
            0.0)
        xo_ref[...] = xo
        r = (jnp.dot(dsn_ref[...], xo, preferred_element_type=jnp.float32)
             + dsnsc_ref[0, 0])
        rm = jnp.mean(r)
        rv = jnp.mean((r - rm) * (r - rm))
        r0_ref[...] = jnp.maximum(
            dsnsc_ref[0, 2] * (r - rm) / jnp.sqrt(rv + 1e-5) + dsnsc_ref[0, 1],
            0.0)


def kernel(feat, Wq, Wk, Wv, conv_w, conv_b, pm_w1, pm_w2, catt, catt1,
           back_w, mlp_w, mlp_b, mlp_g, mlp_beta, dsn_w, dsn_b, dsn_g,
           dsn_beta, H, W):
    x = feat.reshape(C, HW)
    cwb = jnp.stack([conv_w[0, 0, 0, 0], conv_b[0]]).reshape(1, 2)
    pm1t = pm_w1.T
    dsnsc = jnp.stack([dsn_b[0], dsn_beta[0], dsn_g[0]]).reshape(1, 3)

    full = lambda s: pl.BlockSpec(s, lambda i: tuple(0 for _ in s))
    xo, r0 = pl.pallas_call(
        _fused_body,
        grid=(NBLK,),
        in_specs=[
            full((C, HW)),
            full((CIM, C)),
            full((CIM, C)),
            full((CIM, C)),
            full((1, 2)),
            full((TOP, 2 * TOP)),
            full((TOP, 2 * TOP)),
            full((C, CIM)),
            full((C, C)),
            full((C, C)),
            full((C, 1)),
            full((C, 1)),
            full((C, 1)),
            full((C, 1)),
            full((C, 1)),
            full((1, C)),
            full((1, 3)),
        ],
        out_specs=[full((C, HW)), full((1, HW))],
        out_shape=[jax.ShapeDtypeStruct((C, HW), jnp.float32),
                   jax.ShapeDtypeStruct((1, HW), jnp.float32)],
        scratch_shapes=[
            pltpu.VMEM((CIM, HW), jnp.float32),
            pltpu.VMEM((CIM, HW), jnp.float32),
            pltpu.VMEM((CIM, HW), jnp.float32),
            pltpu.VMEM((C, HW), jnp.float32),
        ],
    )(x, Wq, Wk, Wv, cwb, pm1t, pm_w2, back_w, mlp_w[:, :C], mlp_w[:, C:],
      mlp_b.reshape(C, 1), mlp_g.reshape(C, 1), mlp_beta.reshape(C, 1),
      catt.reshape(C, 1), catt1.reshape(C, 1), dsn_w, dsnsc)

    h = feat.shape[2]
    w_ = feat.shape[3]
    return (xo.reshape(1, C, h, w_), r0.reshape(1, 1, h, w_))
